# baseline (speedup 1.0000x reference)
import functools

import jax
import jax.numpy as jnp
from jax import lax
from jax.experimental import pallas as pl
from jax.experimental.pallas import tpu as pltpu
from jax.experimental.pallas import tpu_sc as plsc

_NUM_PATCHES = 2048
_NC = 2
_NS = 16
_CH = 64


def _cast_body(x_ref, o_ref):
    o_ref[...] = x_ref[...].astype(jnp.bfloat16)


def _cast(bytes):
    B, S, D = bytes.shape
    ST = 1024
    ns = S // ST
    return pl.pallas_call(
        _cast_body,
        grid=(B * ns,),
        in_specs=[pl.BlockSpec((1, ST, D), lambda i: (i // ns, i % ns, 0))],
        out_specs=pl.BlockSpec((1, ST, D), lambda i: (i // ns, i % ns, 0)),
        out_shape=jax.ShapeDtypeStruct((B, S, D), jnp.bfloat16),
    )(bytes)


_CH = 16
_PPT = 256
_PHP = 128


def _sc_body(bytesf_hbm, idsf_hbm, sumsf_hbm,
             acc_v, rows0_v, rows1_v, idbuf_v, idx0_v, idx1_v,
             isem0, rsem0, isem1, rsem1, *, B, S, D, NP):
    c = lax.axis_index("c")
    s = lax.axis_index("s")
    w = c * _NS + s
    tiles_per_row = NP // _PPT
    row = w // tiles_per_row
    p0 = (w % tiles_per_row) * _PPT
    rbase = pl.multiple_of(row * S, 1024)

    def cnt_chunk(cb, carry):
        alo, amid, ahi = carry
        pltpu.sync_copy(idsf_hbm.at[pl.ds(rbase + cb * 1024, 1024)], idbuf_v)

        def cnt_vec(j, carry2):
            alo2, amid2, ahi2 = carry2
            v = idbuf_v[pl.ds(j * 16, 16)]
            one = jnp.ones((16,), jnp.int32)
            zero = jnp.zeros((16,), jnp.int32)
            alo2 = alo2 + jnp.where(v < p0, one, zero)
            amid2 = amid2 + jnp.where(v < p0 + _PHP, one, zero)
            ahi2 = ahi2 + jnp.where(v < p0 + _PPT, one, zero)
            return alo2, amid2, ahi2

        return lax.fori_loop(0, 1024 // 16, cnt_vec, (alo, amid, ahi))

    z16 = jnp.zeros((16,), jnp.int32)
    alo, amid, ahi = lax.fori_loop(0, S // 1024, cnt_chunk, (z16, z16, z16))
    lo = alo[0]
    mid = amid[0]
    hi = ahi[0]
    for l in range(1, 16):
        lo = lo + alo[l]
        mid = mid + amid[l]
        hi = hi + ahi[l]

    z16f = jnp.zeros((16,), jnp.float32)
    NW = (_PHP + 1) * D // 256

    rows_bufs = (rows0_v, rows1_v)
    idx_bufs = (idx0_v, idx1_v)
    sems = ((isem0, rsem0), (isem1, rsem1))

    def pass_q(q, _):
        pq = p0 + q * _PHP
        b0 = jnp.where(q == 0, lo, mid)
        b1 = jnp.where(q == 0, mid, hi)

        @plsc.parallel_loop(0, (_PHP + 1) * D, step=16, unroll=16)
        def _(i):
            acc_v[pl.ds(i, 16)] = z16f

        b016 = (b0 // 16) * 16
        ntp = b1 - b016
        nch = (ntp + _CH - 1) // _CH

        def t0c_of(k):
            return pl.multiple_of(jnp.minimum(b016 + k * _CH, S - _CH), 16)

        def start(k, slot):
            t0c = t0c_of(k)
            pltpu.async_copy(idsf_hbm.at[pl.ds(rbase + t0c, _CH)],
                             idx_bufs[slot], sems[slot][0])
            pltpu.async_copy(bytesf_hbm.at[pl.ds((rbase + t0c) * D, _CH * D)],
                             rows_bufs[slot], sems[slot][1])

        def wait(k, slot):
            t0c = t0c_of(k)
            pltpu.make_async_copy(idsf_hbm.at[pl.ds(rbase + t0c, _CH)],
                                  idx_bufs[slot], sems[slot][0]).wait()
            pltpu.make_async_copy(bytesf_hbm.at[pl.ds((rbase + t0c) * D, _CH * D)],
                                  rows_bufs[slot], sems[slot][1]).wait()

        def process(k, slot):
            t0c = t0c_of(k)
            off = b016 + k * _CH - t0c
            idx_v = idx_bufs[slot]
            rows_v = rows_bufs[slot]
            for g in range(_CH // 16):
                idv = idx_v[pl.ds(g * 16, 16)]
                for l in range(16):
                    pos = k * _CH + g * 16 + l
                    rel0 = idv[l] - pq
                    ok = (pos < ntp) & (rel0 >= 0) & (rel0 < _PHP)
                    rel = jnp.where(ok, rel0, _PHP)
                    bs = rel * D
                    bg = (off + g * 16 + l) * D

                    @plsc.parallel_loop(0, D, step=16, unroll=8)
                    def _(j):
                        plsc.addupdate(acc_v.at[pl.ds(bs + j, 16)],
                                       rows_v[pl.ds(bg + j, 16)])

        @pl.when(nch > 0)
        def _():
            start(0, 0)

        def loop2(k2, _):
            k = k2 * 2

            @pl.when(k < nch)
            def _():
                wait(k, 0)

                @pl.when(k + 1 < nch)
                def _():
                    start(k + 1, 1)

                process(k, 0)

            @pl.when(k + 1 < nch)
            def _():
                wait(k + 1, 1)

                @pl.when(k + 2 < nch)
                def _():
                    start(k + 2, 0)

                process(k + 1, 1)

            return 0

        lax.fori_loop(0, (nch + 1) // 2, loop2, 0)

        pltpu.sync_copy(acc_v.at[pl.ds(0, _PHP * D)],
                        sumsf_hbm.at[pl.ds((row * NP + pq) * D, _PHP * D)])
        return 0

    lax.fori_loop(0, 2, pass_q, 0)


def _sc_segsum(bytes, patch_ids):
    B, S, D = bytes.shape
    NP = _NUM_PATCHES
    body = functools.partial(_sc_body, B=B, S=S, D=D, NP=NP)
    f = pl.kernel(
        body,
        out_type=jax.ShapeDtypeStruct((B * NP * D,), jnp.float32),
        mesh=plsc.VectorSubcoreMesh(core_axis_name="c", subcore_axis_name="s"),
        compiler_params=pltpu.CompilerParams(needs_layout_passes=False),
        scratch_types=[
            pltpu.VMEM(((_PHP + 1) * D,), jnp.float32),
            pltpu.VMEM((_CH * D,), jnp.float32),
            pltpu.VMEM((_CH * D,), jnp.float32),
            pltpu.VMEM((1024,), jnp.int32),
            pltpu.VMEM((_CH,), jnp.int32),
            pltpu.VMEM((_CH,), jnp.int32),
            pltpu.SemaphoreType.DMA,
            pltpu.SemaphoreType.DMA,
            pltpu.SemaphoreType.DMA,
            pltpu.SemaphoreType.DMA,
        ],
    )
    sumsf = f(bytes.reshape(-1), patch_ids.reshape(-1).astype(jnp.int32))
    return sumsf.reshape(B, NP, D)


def _proj_body(sums_ref, ids_ref, w_ref, b_ref, o_ref, *, PB, S):
    p = pl.program_id(1)
    p0 = p * PB

    def cnt_chunk(cb, cnt):
        ids = ids_ref[0, 0, pl.ds(cb * 1024, 1024)]
        patches = jax.lax.broadcasted_iota(jnp.int32, (PB, 1024), 0) + p0
        oh = (patches == ids[None, :]).astype(jnp.float32)
        return cnt + jnp.sum(oh, axis=1)

    cnt = lax.fori_loop(0, S // 1024, cnt_chunk, jnp.zeros((PB,), jnp.float32))
    cnt = jnp.maximum(cnt, 1.0)[:, None]
    mean = (sums_ref[0].astype(jnp.float32) / cnt).astype(jnp.bfloat16).astype(jnp.float32)
    o_ref[0] = lax.dot_general(
        mean, w_ref[...], (((1,), (0,)), ((), ())),
        preferred_element_type=jnp.float32) + b_ref[0][None, :]


def _proj(sums, ids3, W, b):
    B, NP, D = sums.shape
    S = ids3.shape[2]
    GD = W.shape[1]
    PB = 512
    npb = NP // PB
    body = functools.partial(_proj_body, PB=PB, S=S)
    return pl.pallas_call(
        body,
        grid=(B, npb),
        in_specs=[
            pl.BlockSpec((1, PB, D), lambda bb, pp: (bb, pp, 0)),
            pl.BlockSpec((1, 1, S), lambda bb, pp: (bb, 0, 0)),
            pl.BlockSpec((D, GD), lambda bb, pp: (0, 0)),
            pl.BlockSpec((1, GD), lambda bb, pp: (0, 0)),
        ],
        out_specs=pl.BlockSpec((1, PB, GD), lambda bb, pp: (bb, pp, 0)),
        out_shape=jax.ShapeDtypeStruct((B, NP, GD), jnp.float32),
    )(sums, ids3, W, b.reshape(1, GD))


def kernel(bytes, patch_ids, W, b):
    B, S, D = bytes.shape
    be = _cast(bytes)
    sums = _sc_segsum(bytes, patch_ids)
    pe = _proj(sums, patch_ids.reshape(B, 1, S).astype(jnp.int32), W, b)
    return (be, pe)


# speedup vs baseline: 1.0995x; 1.0995x over previous
import functools

import jax
import jax.numpy as jnp
from jax import lax
from jax.experimental import pallas as pl
from jax.experimental.pallas import tpu as pltpu
from jax.experimental.pallas import tpu_sc as plsc

_NUM_PATCHES = 2048
_NC = 2
_NS = 16
_CH = 64


def _cast_body(x_ref, o_ref):
    o_ref[...] = x_ref[...].astype(jnp.bfloat16)


def _cast(bytes):
    B, S, D = bytes.shape
    ST = 1024
    ns = S // ST
    return pl.pallas_call(
        _cast_body,
        grid=(B * ns,),
        in_specs=[pl.BlockSpec((1, ST, D), lambda i: (i // ns, i % ns, 0))],
        out_specs=pl.BlockSpec((1, ST, D), lambda i: (i // ns, i % ns, 0)),
        out_shape=jax.ShapeDtypeStruct((B, S, D), jnp.bfloat16),
    )(bytes)


_CH = 16
_PPT = 256
_PHP = 128


def _sc_body(bytesf_hbm, idsf_hbm, sumsf_hbm, cntsf_hbm,
             acc_v, cnt_v, rows0_v, rows1_v, idbuf_v, idx0_v, idx1_v,
             isem0, rsem0, isem1, rsem1, *, B, S, D, NP):
    c = lax.axis_index("c")
    s = lax.axis_index("s")
    w = c * _NS + s
    tiles_per_row = NP // _PPT
    row = w // tiles_per_row
    p0 = (w % tiles_per_row) * _PPT
    rbase = pl.multiple_of(row * S, 1024)

    def cnt_chunk(cb, carry):
        alo, amid, ahi = carry
        pltpu.sync_copy(idsf_hbm.at[pl.ds(rbase + cb * 1024, 1024)], idbuf_v)

        def cnt_vec(j, carry2):
            alo2, amid2, ahi2 = carry2
            v = idbuf_v[pl.ds(j * 16, 16)]
            one = jnp.ones((16,), jnp.int32)
            zero = jnp.zeros((16,), jnp.int32)
            alo2 = alo2 + jnp.where(v < p0, one, zero)
            amid2 = amid2 + jnp.where(v < p0 + _PHP, one, zero)
            ahi2 = ahi2 + jnp.where(v < p0 + _PPT, one, zero)
            return alo2, amid2, ahi2

        return lax.fori_loop(0, 1024 // 16, cnt_vec, (alo, amid, ahi))

    z16 = jnp.zeros((16,), jnp.int32)
    alo, amid, ahi = lax.fori_loop(0, S // 1024, cnt_chunk, (z16, z16, z16))
    lo = alo[0]
    mid = amid[0]
    hi = ahi[0]
    for l in range(1, 16):
        lo = lo + alo[l]
        mid = mid + amid[l]
        hi = hi + ahi[l]

    z16f = jnp.zeros((16,), jnp.float32)
    NW = (_PHP + 1) * D // 256

    rows_bufs = (rows0_v, rows1_v)
    idx_bufs = (idx0_v, idx1_v)
    sems = ((isem0, rsem0), (isem1, rsem1))

    def pass_q(q, _):
        pq = p0 + q * _PHP
        b0 = jnp.where(q == 0, lo, mid)
        b1 = jnp.where(q == 0, mid, hi)

        @plsc.parallel_loop(0, (_PHP + 1) * D, step=16, unroll=16)
        def _(i):
            acc_v[pl.ds(i, 16)] = z16f

        @plsc.parallel_loop(0, 144, step=16)
        def _(i):
            cnt_v[pl.ds(i, 16)] = z16f

        b016 = (b0 // 16) * 16
        ntp = b1 - b016
        nch = (ntp + _CH - 1) // _CH

        def t0c_of(k):
            return pl.multiple_of(jnp.minimum(b016 + k * _CH, S - _CH), 16)

        def start(k, slot):
            t0c = t0c_of(k)
            pltpu.async_copy(idsf_hbm.at[pl.ds(rbase + t0c, _CH)],
                             idx_bufs[slot], sems[slot][0])
            pltpu.async_copy(bytesf_hbm.at[pl.ds((rbase + t0c) * D, _CH * D)],
                             rows_bufs[slot], sems[slot][1])

        def wait(k, slot):
            t0c = t0c_of(k)
            pltpu.make_async_copy(idsf_hbm.at[pl.ds(rbase + t0c, _CH)],
                                  idx_bufs[slot], sems[slot][0]).wait()
            pltpu.make_async_copy(bytesf_hbm.at[pl.ds((rbase + t0c) * D, _CH * D)],
                                  rows_bufs[slot], sems[slot][1]).wait()

        def process(k, slot):
            t0c = t0c_of(k)
            off = b016 + k * _CH - t0c
            idx_v = idx_bufs[slot]
            rows_v = rows_bufs[slot]
            ones16f = jnp.ones((16,), jnp.float32)
            iota16 = lax.broadcasted_iota(jnp.int32, (16,), 0)
            for g in range(_CH // 16):
                idv = idx_v[pl.ds(g * 16, 16)]
                posv = iota16 + (k * _CH + g * 16)
                rel0v = idv - pq
                okv = (posv < ntp) & (rel0v >= 0) & (rel0v < _PHP)
                relv = jnp.where(okv, rel0v, _PHP)
                plsc.addupdate_scatter(cnt_v, [relv], ones16f)
                for l in range(16):
                    pos = k * _CH + g * 16 + l
                    rel0 = idv[l] - pq
                    ok = (pos < ntp) & (rel0 >= 0) & (rel0 < _PHP)
                    rel = jnp.where(ok, rel0, _PHP)
                    bs = rel * D
                    bg = (off + g * 16 + l) * D

                    @plsc.parallel_loop(0, D, step=16, unroll=8)
                    def _(j):
                        plsc.addupdate(acc_v.at[pl.ds(bs + j, 16)],
                                       rows_v[pl.ds(bg + j, 16)])

        @pl.when(nch > 0)
        def _():
            start(0, 0)

        def loop2(k2, _):
            k = k2 * 2

            @pl.when(k < nch)
            def _():
                wait(k, 0)

                @pl.when(k + 1 < nch)
                def _():
                    start(k + 1, 1)

                process(k, 0)

            @pl.when(k + 1 < nch)
            def _():
                wait(k + 1, 1)

                @pl.when(k + 2 < nch)
                def _():
                    start(k + 2, 0)

                process(k + 1, 1)

            return 0

        lax.fori_loop(0, (nch + 1) // 2, loop2, 0)

        pltpu.sync_copy(acc_v.at[pl.ds(0, _PHP * D)],
                        sumsf_hbm.at[pl.ds((row * NP + pq) * D, _PHP * D)])
        pltpu.sync_copy(cnt_v.at[pl.ds(0, _PHP)],
                        cntsf_hbm.at[pl.ds(row * NP + pq, _PHP)])
        return 0

    lax.fori_loop(0, 2, pass_q, 0)


def _sc_segsum(bytes, patch_ids):
    B, S, D = bytes.shape
    NP = _NUM_PATCHES
    body = functools.partial(_sc_body, B=B, S=S, D=D, NP=NP)
    f = pl.kernel(
        body,
        out_type=[jax.ShapeDtypeStruct((B * NP * D,), jnp.float32),
                  jax.ShapeDtypeStruct((B * NP,), jnp.float32)],
        mesh=plsc.VectorSubcoreMesh(core_axis_name="c", subcore_axis_name="s"),
        compiler_params=pltpu.CompilerParams(needs_layout_passes=False),
        scratch_types=[
            pltpu.VMEM(((_PHP + 1) * D,), jnp.float32),
            pltpu.VMEM((144,), jnp.float32),
            pltpu.VMEM((_CH * D,), jnp.float32),
            pltpu.VMEM((_CH * D,), jnp.float32),
            pltpu.VMEM((1024,), jnp.int32),
            pltpu.VMEM((_CH,), jnp.int32),
            pltpu.VMEM((_CH,), jnp.int32),
            pltpu.SemaphoreType.DMA,
            pltpu.SemaphoreType.DMA,
            pltpu.SemaphoreType.DMA,
            pltpu.SemaphoreType.DMA,
        ],
    )
    sumsf, cntsf = f(bytes.reshape(-1), patch_ids.reshape(-1).astype(jnp.int32))
    return sumsf.reshape(B, NP, D), cntsf.reshape(B, 1, NP)


def _proj_body(sums_ref, cnts_ref, w_ref, b_ref, o_ref):
    cnt = jnp.maximum(cnts_ref[0, 0], 1.0)[:, None]
    mean = (sums_ref[0] / cnt).astype(jnp.bfloat16).astype(jnp.float32)
    o_ref[0] = lax.dot_general(
        mean, w_ref[...], (((1,), (0,)), ((), ())),
        preferred_element_type=jnp.float32) + b_ref[0][None, :]


def _proj(sums, cnts3, W, b):
    B, NP, D = sums.shape
    GD = W.shape[1]
    PB = 512
    npb = NP // PB
    return pl.pallas_call(
        _proj_body,
        grid=(B, npb),
        in_specs=[
            pl.BlockSpec((1, PB, D), lambda bb, pp: (bb, pp, 0)),
            pl.BlockSpec((1, 1, PB), lambda bb, pp: (bb, 0, pp)),
            pl.BlockSpec((D, GD), lambda bb, pp: (0, 0)),
            pl.BlockSpec((1, GD), lambda bb, pp: (0, 0)),
        ],
        out_specs=pl.BlockSpec((1, PB, GD), lambda bb, pp: (bb, pp, 0)),
        out_shape=jax.ShapeDtypeStruct((B, NP, GD), jnp.float32),
    )(sums, cnts3, W, b.reshape(1, GD))


def kernel(bytes, patch_ids, W, b):
    B, S, D = bytes.shape
    be = _cast(bytes)
    sums, cnts3 = _sc_segsum(bytes, patch_ids)
    pe = _proj(sums, cnts3, W, b)
    return (be, pe)
